# baseline (speedup 1.0000x reference)
import functools

import jax
import jax.numpy as jnp
from jax import lax
from jax.experimental import pallas as pl
from jax.experimental.pallas import tpu as pltpu
from jax.experimental.pallas import tpu_sc as plsc

N, E, B, H = 10000, 320000, 256, 128

LN = 16
NC, NS = 2, 16
NW = NC * NS
EPW = E // NW
CHUNK = 80
NCH = EPW // CHUNK
CH_ELEMS = CHUNK * H

_mesh = plsc.VectorSubcoreMesh(core_axis_name="c", subcore_axis_name="s")


@functools.partial(
    pl.kernel,
    out_type=(jax.ShapeDtypeStruct((NW, B * H), jnp.float32),
              jax.ShapeDtypeStruct((NW, B), jnp.float32)),
    mesh=_mesh,
    scratch_types=[
        pltpu.VMEM((N,), jnp.int32),
        pltpu.VMEM((EPW,), jnp.int32),
        pltpu.VMEM((EPW,), jnp.int32),
        pltpu.VMEM((2, CH_ELEMS), jnp.float32),
        pltpu.VMEM((B * H,), jnp.float32),
        pltpu.VMEM((B * LN,), jnp.float32),
        pltpu.VMEM((B,), jnp.float32),
        pltpu.SemaphoreType.DMA,
        pltpu.SemaphoreType.DMA,
    ],
    compiler_params=pltpu.CompilerParams(needs_layout_passes=False),
)
def _sc_edge_pool(row_hbm, batch_hbm, edge_hbm, sums_hbm, cnts_hbm,
                  batch_v, row_v, seg_v, ebuf, acc_v, cnt_v, cntc_v,
                  sem0, sem1):
    wid = lax.axis_index("s") * NC + lax.axis_index("c")
    base = wid * EPW

    pltpu.sync_copy(batch_hbm, batch_v)
    pltpu.sync_copy(row_hbm.at[pl.ds(base, EPW)], row_v)

    zeros16 = jnp.zeros((LN,), jnp.float32)
    ones16 = jnp.ones((LN,), jnp.float32)

    def _zero_acc(i, carry):
        for k in range(16):
            acc_v[pl.ds(i * 256 + k * LN, LN)] = zeros16
        return carry
    lax.fori_loop(0, (B * H) // 256, _zero_acc, 0)

    def _zero_cnt(i, carry):
        for k in range(16):
            cnt_v[pl.ds(i * 256 + k * LN, LN)] = zeros16
        return carry
    lax.fori_loop(0, (B * LN) // 256, _zero_cnt, 0)

    def _gather_seg(g, carry):
        for k in range(5):
            off = g * 80 + k * LN
            r16 = row_v[pl.ds(off, LN)]
            seg_v[pl.ds(off, LN)] = plsc.load_gather(batch_v, [r16])
        return carry
    lax.fori_loop(0, EPW // 80, _gather_seg, 0)

    def _chunk_src(c):
        return edge_hbm.at[pl.ds((base + c * CHUNK) * H, CH_ELEMS)]

    pltpu.async_copy(_chunk_src(0), ebuf.at[0], sem0)
    pltpu.async_copy(_chunk_src(1), ebuf.at[1], sem1)

    def _main(cc, carry):
        for slot in range(2):
            c = cc * 2 + slot
            sem = sem0 if slot == 0 else sem1

            @pl.when(c < NCH)
            def _process():
                pltpu.make_async_copy(_chunk_src(c), ebuf.at[slot], sem).wait()
                cb = c * CHUNK
                for g in range(CHUNK // LN):
                    sv = seg_v[pl.ds(cb + g * LN, LN)]
                    for j in range(LN):
                        e_loc = g * LN + j
                        s = sv[j]
                        soff = s * H
                        for k in range(H // LN):
                            v = ebuf[slot, pl.ds(e_loc * H + k * LN, LN)]
                            plsc.addupdate(acc_v.at[pl.ds(soff + k * LN, LN)], v)
                        plsc.addupdate(cnt_v.at[pl.ds(s * LN, LN)], ones16)

                @pl.when(c + 2 < NCH)
                def _issue_next():
                    pltpu.async_copy(_chunk_src(c + 2), ebuf.at[slot], sem)
        return carry
    lax.fori_loop(0, (NCH + 1) // 2, _main, 0)

    idx0 = lax.iota(jnp.int32, LN) * LN
    def _compact(i, carry):
        cntc_v[pl.ds(i * LN, LN)] = plsc.load_gather(cnt_v, [idx0 + i * (LN * LN)])
        return carry
    lax.fori_loop(0, B // LN, _compact, 0)

    pltpu.sync_copy(acc_v, sums_hbm.at[wid])
    pltpu.sync_copy(cntc_v, cnts_hbm.at[wid])


def _tc_finish_body(ps_ref, pc_ref, x_ref, batch_ref, u_ref, w1_ref, b1_ref,
                    w2_ref, b2_ref, out_ref):
    e_sum = jnp.sum(ps_ref[...], axis=0)
    cnt_row = jnp.sum(pc_ref[...], axis=0, keepdims=True)
    eye = (jax.lax.broadcasted_iota(jnp.int32, (B, B), 0)
           == jax.lax.broadcasted_iota(jnp.int32, (B, B), 1)).astype(jnp.float32)
    dn = (((1,), (1,)), ((), ()))
    cnt_col = jax.lax.dot_general(eye, cnt_row, dn,
                                  preferred_element_type=jnp.float32)
    e_mean = e_sum / jnp.maximum(cnt_col, 1.0)

    b_iota = jax.lax.broadcasted_iota(jnp.int32, (B, N), 0)
    hist_col = jnp.sum(jnp.equal(batch_ref[...], b_iota).astype(jnp.float32),
                       axis=1, keepdims=True)
    tri = (jax.lax.broadcasted_iota(jnp.int32, (B, B), 0)
           > jax.lax.broadcasted_iota(jnp.int32, (B, B), 1)).astype(jnp.float32)
    starts_col = jnp.dot(tri, hist_col, preferred_element_type=jnp.float32)
    s_col = starts_col.astype(jnp.int32)
    h_col = hist_col.astype(jnp.int32)
    n_iota = jax.lax.broadcasted_iota(jnp.int32, (B, N), 1)
    maskx = ((n_iota >= s_col) & (n_iota < s_col + h_col)).astype(jnp.float32)
    sum_x = jnp.dot(maskx, x_ref[...], preferred_element_type=jnp.float32)
    x_mean = sum_x / jnp.maximum(hist_col, 1.0)

    cat = jnp.concatenate([u_ref[...], x_mean, e_mean], axis=1)
    h1 = jax.lax.dot_general(cat, w1_ref[...], dn,
                             preferred_element_type=jnp.float32) + b1_ref[...]
    h1 = jnp.maximum(h1, 0.0)
    out_ref[...] = jax.lax.dot_general(h1, w2_ref[...], dn,
                                       preferred_element_type=jnp.float32) + b2_ref[...]


def _tc_finish(part_sums, part_cnts, x, batch2, u, W1, b1r, W2, b2r):
    return pl.pallas_call(
        _tc_finish_body,
        grid=(1,),
        in_specs=[
            pl.BlockSpec((NW, B, H), lambda i: (0, 0, 0)),
            pl.BlockSpec((NW, B), lambda i: (0, 0)),
            pl.BlockSpec((N, H), lambda i: (0, 0)),
            pl.BlockSpec((1, N), lambda i: (0, 0)),
            pl.BlockSpec((B, H), lambda i: (0, 0)),
            pl.BlockSpec((H, 3 * H), lambda i: (0, 0)),
            pl.BlockSpec((1, H), lambda i: (0, 0)),
            pl.BlockSpec((H, H), lambda i: (0, 0)),
            pl.BlockSpec((1, H), lambda i: (0, 0)),
        ],
        out_specs=pl.BlockSpec((B, H), lambda i: (0, 0)),
        out_shape=jax.ShapeDtypeStruct((B, H), jnp.float32),
        compiler_params=pltpu.CompilerParams(
            dimension_semantics=("arbitrary",),
        ),
    )(part_sums, part_cnts, x, batch2, u, W1, b1r, W2, b2r)


def kernel(x, edge_index, edge_attr, u, batch, W1, b1, W2, b2):
    row = edge_index[0]
    part_sums, part_cnts = _sc_edge_pool(row, batch, edge_attr.reshape(E * H))
    return _tc_finish(part_sums.reshape(NW, B, H), part_cnts, x,
                      batch.reshape(1, N), u, W1, b1.reshape(1, H),
                      W2, b2.reshape(1, H))


# speedup vs baseline: 3.4183x; 3.4183x over previous
import functools

import jax
import jax.numpy as jnp
from jax import lax
from jax.experimental import pallas as pl
from jax.experimental.pallas import tpu as pltpu
from jax.experimental.pallas import tpu_sc as plsc

N, E, B, H = 10000, 320000, 256, 128

LN = 16
NC, NS = 2, 16
NW = NC * NS
EPW = E // NW
CHUNK = 80
NCH = EPW // CHUNK
NBUF = 4

_mesh = plsc.VectorSubcoreMesh(core_axis_name="c", subcore_axis_name="s")


@functools.partial(
    pl.kernel,
    out_type=(jax.ShapeDtypeStruct((NC, B, H), jnp.float32),
              jax.ShapeDtypeStruct((NW, B * LN), jnp.float32)),
    mesh=_mesh,
    scratch_types=[
        pltpu.VMEM((N,), jnp.int32),
        pltpu.VMEM((EPW,), jnp.int32),
        pltpu.VMEM((NBUF, CHUNK, H), jnp.float32),
        pltpu.VMEM((NBUF, CHUNK), jnp.int32),
        pltpu.VMEM_SHARED((B, H), jnp.float32),
        pltpu.VMEM((B * LN,), jnp.float32),
        [pltpu.SemaphoreType.DMA] * NBUF,
        [pltpu.SemaphoreType.DMA] * NBUF,
    ],
    compiler_params=pltpu.CompilerParams(needs_layout_passes=False),
)
def _sc_edge_pool(row_hbm, batch_hbm, edge_hbm, zsum_hbm,
                  sums_hbm, cnts_hbm,
                  batch_v, row_v, ebuf, idx_v, acc_v, cnt_v,
                  dsem, ssem):
    sid = lax.axis_index("s")
    cid = lax.axis_index("c")
    wid = sid * NC + cid
    base = wid * EPW

    pltpu.sync_copy(batch_hbm, batch_v)
    pltpu.sync_copy(row_hbm.at[pl.ds(base, EPW)], row_v)

    @pl.when(sid == 0)
    def _zero_shared():
        pltpu.sync_copy(zsum_hbm, acc_v)

    zeros16 = jnp.zeros((LN,), jnp.float32)
    ones16 = jnp.ones((LN,), jnp.float32)

    def _zero_cnt(i, carry):
        for k in range(16):
            cnt_v[pl.ds(i * 256 + k * LN, LN)] = zeros16
        return carry
    lax.fori_loop(0, (B * LN) // 256, _zero_cnt, 0)

    plsc.subcore_barrier()

    def _chunk_src(c):
        return edge_hbm.at[pl.ds(base + c * CHUNK, CHUNK), :]

    def _fill_idx(c, s):
        for k in range(CHUNK // LN):
            r16 = row_v[pl.ds(c * CHUNK + k * LN, LN)]
            idx_v[s, pl.ds(k * LN, LN)] = plsc.load_gather(batch_v, [r16])

    def _scatter_desc(s):
        return pltpu.make_async_copy(ebuf.at[s], acc_v.at[idx_v.at[s]],
                                     ssem[s])

    pltpu.async_copy(_chunk_src(0), ebuf.at[0], dsem[0])
    pltpu.async_copy(_chunk_src(1), ebuf.at[1], dsem[1])

    def _turn(cc, carry):
        for s in range(NBUF):
            c = cc * NBUF + s

            @pl.when(c < NCH)
            def _process():
                pltpu.make_async_copy(_chunk_src(c), ebuf.at[s], dsem[s]).wait()
                _fill_idx(c, s)
                pltpu.async_copy(ebuf.at[s], acc_v.at[idx_v.at[s]], ssem[s],
                                 add=True)
                for k in range(CHUNK // LN):
                    sv = idx_v[s, pl.ds(k * LN, LN)]
                    for j in range(LN):
                        sg = sv[j]
                        plsc.addupdate(cnt_v.at[pl.ds(sg * LN, LN)], ones16)

            sp = (s + 2) % NBUF

            @pl.when(c + 2 < NCH)
            def _prefetch():
                @pl.when(c >= 2)
                def _drain_prev():
                    _scatter_desc(sp).wait()
                pltpu.async_copy(_chunk_src(c + 2), ebuf.at[sp], dsem[sp])
        return carry
    lax.fori_loop(0, (NCH + NBUF - 1) // NBUF, _turn, 0)

    for cf in range(NCH - NBUF, NCH):
        _scatter_desc(cf % NBUF).wait()
    plsc.subcore_barrier()

    @pl.when(sid == 0)
    def _out_sums():
        pltpu.sync_copy(acc_v, sums_hbm.at[cid])
    pltpu.sync_copy(cnt_v, cnts_hbm.at[wid])


def _tc_finish_body(ps_ref, pc_ref, x_ref, batch_ref, u_ref, w1_ref, b1_ref,
                    w2_ref, b2_ref, out_ref):
    dn = (((1,), (1,)), ((), ()))
    e_sum = jnp.sum(ps_ref[...], axis=0)
    cnt_col = jnp.sum(pc_ref[...], axis=0)[:, 0:1]
    e_mean = e_sum / jnp.maximum(cnt_col, 1.0)

    b_iota = jax.lax.broadcasted_iota(jnp.int32, (B, N), 0)
    hist_col = jnp.sum(jnp.equal(batch_ref[...], b_iota).astype(jnp.float32),
                       axis=1, keepdims=True)
    tri = (jax.lax.broadcasted_iota(jnp.int32, (B, B), 0)
           > jax.lax.broadcasted_iota(jnp.int32, (B, B), 1)).astype(jnp.float32)
    starts_col = jnp.dot(tri, hist_col, preferred_element_type=jnp.float32)
    s_col = starts_col.astype(jnp.int32)
    h_col = hist_col.astype(jnp.int32)
    n_iota = jax.lax.broadcasted_iota(jnp.int32, (B, N), 1)
    maskx = ((n_iota >= s_col) & (n_iota < s_col + h_col)).astype(jnp.float32)
    sum_x = jnp.dot(maskx, x_ref[...], preferred_element_type=jnp.float32)
    x_mean = sum_x / jnp.maximum(hist_col, 1.0)

    cat = jnp.concatenate([u_ref[...], x_mean, e_mean], axis=1)
    h1 = jax.lax.dot_general(cat, w1_ref[...], dn,
                             preferred_element_type=jnp.float32) + b1_ref[...]
    h1 = jnp.maximum(h1, 0.0)
    out_ref[...] = jax.lax.dot_general(h1, w2_ref[...], dn,
                                       preferred_element_type=jnp.float32) + b2_ref[...]


def _tc_finish(part_sums, part_cnts, x, batch2, u, W1, b1r, W2, b2r):
    return pl.pallas_call(
        _tc_finish_body,
        grid=(1,),
        in_specs=[
            pl.BlockSpec((NC, B, H), lambda i: (0, 0, 0)),
            pl.BlockSpec((NW, B, LN), lambda i: (0, 0, 0)),
            pl.BlockSpec((N, H), lambda i: (0, 0)),
            pl.BlockSpec((1, N), lambda i: (0, 0)),
            pl.BlockSpec((B, H), lambda i: (0, 0)),
            pl.BlockSpec((H, 3 * H), lambda i: (0, 0)),
            pl.BlockSpec((1, H), lambda i: (0, 0)),
            pl.BlockSpec((H, H), lambda i: (0, 0)),
            pl.BlockSpec((1, H), lambda i: (0, 0)),
        ],
        out_specs=pl.BlockSpec((B, H), lambda i: (0, 0)),
        out_shape=jax.ShapeDtypeStruct((B, H), jnp.float32),
        compiler_params=pltpu.CompilerParams(
            dimension_semantics=("arbitrary",),
        ),
    )(part_sums, part_cnts, x, batch2, u, W1, b1r, W2, b2r)


def kernel(x, edge_index, edge_attr, u, batch, W1, b1, W2, b2):
    row = edge_index[0]
    zsum = jnp.zeros((B, H), jnp.float32)
    part_sums, part_cnts = _sc_edge_pool(row, batch, edge_attr, zsum)
    return _tc_finish(part_sums, part_cnts.reshape(NW, B, LN), x,
                      batch.reshape(1, N), u, W1, b1.reshape(1, H),
                      W2, b2.reshape(1, H))


# speedup vs baseline: 3.5447x; 1.0370x over previous
import functools

import jax
import jax.numpy as jnp
from jax import lax
from jax.experimental import pallas as pl
from jax.experimental.pallas import tpu as pltpu
from jax.experimental.pallas import tpu_sc as plsc

N, E, B, H = 10000, 320000, 256, 128

LN = 16
NC, NS = 2, 16
NW = NC * NS
EPW = E // NW
CHUNK = 80
NCH = EPW // CHUNK
NBUF = 4

_mesh = plsc.VectorSubcoreMesh(core_axis_name="c", subcore_axis_name="s")


@functools.partial(
    pl.kernel,
    out_type=(jax.ShapeDtypeStruct((NC, B, H), jnp.float32),
              jax.ShapeDtypeStruct((NW, B * LN), jnp.float32)),
    mesh=_mesh,
    scratch_types=[
        pltpu.VMEM((N,), jnp.int32),
        pltpu.VMEM((EPW,), jnp.int32),
        pltpu.VMEM((NBUF, CHUNK, H), jnp.float32),
        pltpu.VMEM((NBUF, CHUNK), jnp.int32),
        pltpu.VMEM_SHARED((B, H), jnp.float32),
        pltpu.VMEM((B * LN,), jnp.float32),
        [pltpu.SemaphoreType.DMA] * NBUF,
        [pltpu.SemaphoreType.DMA] * NBUF,
    ],
    compiler_params=pltpu.CompilerParams(needs_layout_passes=False),
)
def _sc_edge_pool(row_hbm, batch_hbm, edge_hbm, zsum_hbm,
                  sums_hbm, cnts_hbm,
                  batch_v, row_v, ebuf, idx_v, acc_v, cnt_v,
                  dsem, ssem):
    sid = lax.axis_index("s")
    cid = lax.axis_index("c")
    wid = sid * NC + cid
    base = wid * EPW

    pltpu.sync_copy(batch_hbm, batch_v)
    pltpu.sync_copy(row_hbm.at[pl.ds(base, EPW)], row_v)

    @pl.when(sid == 0)
    def _zero_shared():
        pltpu.sync_copy(zsum_hbm, acc_v)

    zeros16 = jnp.zeros((LN,), jnp.float32)
    ones16 = jnp.ones((LN,), jnp.float32)
    lane_iota = lax.iota(jnp.int32, LN)

    def _zero_cnt(i, carry):
        for k in range(16):
            cnt_v[pl.ds(i * 256 + k * LN, LN)] = zeros16
        return carry
    lax.fori_loop(0, (B * LN) // 256, _zero_cnt, 0)

    plsc.subcore_barrier()

    def _chunk_src(c):
        return edge_hbm.at[pl.ds(base + c * CHUNK, CHUNK), :]

    def _fill_idx(c, s):
        for k in range(CHUNK // LN):
            r16 = row_v[pl.ds(c * CHUNK + k * LN, LN)]
            idx_v[s, pl.ds(k * LN, LN)] = plsc.load_gather(batch_v, [r16])

    def _scatter_desc(s):
        return pltpu.make_async_copy(ebuf.at[s], acc_v.at[idx_v.at[s]],
                                     ssem[s])

    pltpu.async_copy(_chunk_src(0), ebuf.at[0], dsem[0])
    pltpu.async_copy(_chunk_src(1), ebuf.at[1], dsem[1])

    def _turn(cc, carry):
        for s in range(NBUF):
            c = cc * NBUF + s

            @pl.when(c < NCH)
            def _process():
                pltpu.make_async_copy(_chunk_src(c), ebuf.at[s], dsem[s]).wait()
                _fill_idx(c, s)
                pltpu.async_copy(ebuf.at[s], acc_v.at[idx_v.at[s]], ssem[s],
                                 add=True)
                for k in range(CHUNK // LN):
                    sv = idx_v[s, pl.ds(k * LN, LN)]
                    tgt = sv * LN + lane_iota
                    plsc.addupdate_scatter(cnt_v, [tgt], ones16)

            sp = (s + 2) % NBUF

            @pl.when(c + 2 < NCH)
            def _prefetch():
                @pl.when(c >= 2)
                def _drain_prev():
                    _scatter_desc(sp).wait()
                pltpu.async_copy(_chunk_src(c + 2), ebuf.at[sp], dsem[sp])
        return carry
    lax.fori_loop(0, (NCH + NBUF - 1) // NBUF, _turn, 0)

    for cf in range(NCH - NBUF, NCH):
        _scatter_desc(cf % NBUF).wait()
    plsc.subcore_barrier()

    @pl.when(sid == 0)
    def _out_sums():
        pltpu.sync_copy(acc_v, sums_hbm.at[cid])
    pltpu.sync_copy(cnt_v, cnts_hbm.at[wid])


def _tc_x_body(x_ref, batch_ref, out_ref):
    b_iota = jax.lax.broadcasted_iota(jnp.int32, (B, N), 0)
    hist_col = jnp.sum(jnp.equal(batch_ref[...], b_iota).astype(jnp.float32),
                       axis=1, keepdims=True)
    tri = (jax.lax.broadcasted_iota(jnp.int32, (B, B), 0)
           > jax.lax.broadcasted_iota(jnp.int32, (B, B), 1)).astype(jnp.float32)
    starts_col = jnp.dot(tri, hist_col, preferred_element_type=jnp.float32)
    s_col = starts_col.astype(jnp.int32)
    h_col = hist_col.astype(jnp.int32)
    n_iota = jax.lax.broadcasted_iota(jnp.int32, (B, N), 1)
    maskx = ((n_iota >= s_col) & (n_iota < s_col + h_col)).astype(jnp.float32)
    sum_x = jnp.dot(maskx, x_ref[...], preferred_element_type=jnp.float32)
    out_ref[...] = sum_x / jnp.maximum(hist_col, 1.0)


def _tc_x(x, batch2):
    return pl.pallas_call(
        _tc_x_body,
        grid=(1,),
        in_specs=[
            pl.BlockSpec((N, H), lambda i: (0, 0)),
            pl.BlockSpec((1, N), lambda i: (0, 0)),
        ],
        out_specs=pl.BlockSpec((B, H), lambda i: (0, 0)),
        out_shape=jax.ShapeDtypeStruct((B, H), jnp.float32),
        compiler_params=pltpu.CompilerParams(
            dimension_semantics=("arbitrary",),
        ),
    )(x, batch2)


def _tc_combine_body(ps_ref, pc_ref, xm_ref, u_ref, w1_ref, b1_ref,
                     w2_ref, b2_ref, out_ref):
    dn = (((1,), (1,)), ((), ()))
    e_sum = jnp.sum(ps_ref[...], axis=0)
    cnt_col = jnp.sum(jnp.sum(pc_ref[...], axis=0), axis=1,
                      keepdims=True)
    e_mean = e_sum / jnp.maximum(cnt_col, 1.0)
    cat = jnp.concatenate([u_ref[...], xm_ref[...], e_mean], axis=1)
    h1 = jax.lax.dot_general(cat, w1_ref[...], dn,
                             preferred_element_type=jnp.float32) + b1_ref[...]
    h1 = jnp.maximum(h1, 0.0)
    out_ref[...] = jax.lax.dot_general(h1, w2_ref[...], dn,
                                       preferred_element_type=jnp.float32) + b2_ref[...]


def _tc_combine(part_sums, part_cnts, x_mean, u, W1, b1r, W2, b2r):
    return pl.pallas_call(
        _tc_combine_body,
        grid=(1,),
        in_specs=[
            pl.BlockSpec((NC, B, H), lambda i: (0, 0, 0)),
            pl.BlockSpec((NW, B, LN), lambda i: (0, 0, 0)),
            pl.BlockSpec((B, H), lambda i: (0, 0)),
            pl.BlockSpec((B, H), lambda i: (0, 0)),
            pl.BlockSpec((H, 3 * H), lambda i: (0, 0)),
            pl.BlockSpec((1, H), lambda i: (0, 0)),
            pl.BlockSpec((H, H), lambda i: (0, 0)),
            pl.BlockSpec((1, H), lambda i: (0, 0)),
        ],
        out_specs=pl.BlockSpec((B, H), lambda i: (0, 0)),
        out_shape=jax.ShapeDtypeStruct((B, H), jnp.float32),
        compiler_params=pltpu.CompilerParams(
            dimension_semantics=("arbitrary",),
        ),
    )(part_sums, part_cnts, x_mean, u, W1, b1r, W2, b2r)


def kernel(x, edge_index, edge_attr, u, batch, W1, b1, W2, b2):
    row = edge_index[0]
    zsum = jnp.zeros((B, H), jnp.float32)
    part_sums, part_cnts = _sc_edge_pool(row, batch, edge_attr, zsum)
    x_mean = _tc_x(x, batch.reshape(1, N))
    return _tc_combine(part_sums, part_cnts.reshape(NW, B, LN), x_mean,
                       u, W1, b1.reshape(1, H), W2, b2.reshape(1, H))
